# Initial kernel scaffold; baseline (speedup 1.0000x reference)
import jax
import jax.numpy as jnp
from jax.experimental import pallas as pl


def kernel(x, ln1_g, ln1_b, ln2_g, ln2_b, Wq, bq, Wk, bk, Wv, bv, Wo, bo, Wr, W1, b1, W2, b2):
    raise NotImplementedError("write your pallas kernel here")


# speedup vs baseline: 1.7788x; 1.7788x over previous
import functools

import jax
import jax.numpy as jnp
from jax import lax
from jax.experimental import pallas as pl
from jax.experimental.pallas import tpu as pltpu

D = 768
H = 12
DH = D // H
E = 8
K = 2
F = 3072
SB = 512
BT = 128
FB = 512
NF = F // FB


def _ln_rows(x, g, b):
    m = jnp.mean(x, axis=1, keepdims=True)
    xc = x - m
    v = jnp.mean(xc * xc, axis=1, keepdims=True)
    return xc * lax.rsqrt(v + 1e-5) * g + b


def _qkv_body(x_ref, g_ref, b_ref, wq_ref, bq_ref, wk_ref, bk_ref,
              wv_ref, bv_ref, q_ref, k_ref, v_ref):
    h = _ln_rows(x_ref[...], g_ref[...], b_ref[...]).astype(jnp.bfloat16)
    for w_ref, bb_ref, o_ref in ((wq_ref, bq_ref, q_ref),
                                 (wk_ref, bk_ref, k_ref),
                                 (wv_ref, bv_ref, v_ref)):
        r = jnp.dot(h, w_ref[...].astype(jnp.bfloat16),
                    preferred_element_type=jnp.float32) + bb_ref[...]
        rb = r.astype(jnp.bfloat16)
        for hh in range(H):
            o_ref[hh, :, :] = rb[:, hh * DH:(hh + 1) * DH]


def _qkv_call(x, ln1_g, ln1_b, Wq, bq, Wk, bk, Wv, bv, interpret=False):
    S = x.shape[0]
    hd = jax.ShapeDtypeStruct((H, S, DH), jnp.bfloat16)
    full = lambda shp: pl.BlockSpec(shp, lambda s: (0,) * len(shp))
    return pl.pallas_call(
        _qkv_body,
        grid=(S // SB,),
        in_specs=[
            pl.BlockSpec((SB, D), lambda s: (s, 0)),
            full((1, D)), full((1, D)),
            full((D, D)), full((1, D)),
            full((D, D)), full((1, D)),
            full((D, D)), full((1, D)),
        ],
        out_specs=[pl.BlockSpec((H, SB, DH), lambda s: (0, s, 0))] * 3,
        out_shape=[hd, hd, hd],
        compiler_params=pltpu.CompilerParams(
            dimension_semantics=("parallel",)),
        interpret=interpret,
    )(x, ln1_g, ln1_b, Wq, bq, Wk, bk, Wv, bv)


def _attn_body(x_ref, q_ref, k_ref, v_ref, wo_ref, bo_ref, g2_ref, b2_ref,
               wrt_ref, x2_ref, tb_ref, lgt_ref):
    hh = pl.program_id(1)
    q = q_ref[0]
    k = k_ref[0]
    v = v_ref[0]
    s = lax.dot_general(q, k, (((1,), (1,)), ((), ())),
                        preferred_element_type=jnp.float32) * (DH ** -0.5)
    s = s - jnp.max(s, axis=1, keepdims=True)
    p = jnp.exp(s)
    p = p / jnp.sum(p, axis=1, keepdims=True)
    o = jnp.dot(p.astype(jnp.bfloat16), v, preferred_element_type=jnp.float32)
    op = jnp.dot(o.astype(jnp.bfloat16), wo_ref[...].astype(jnp.bfloat16),
                 preferred_element_type=jnp.float32)

    @pl.when(hh == 0)
    def _():
        x2_ref[...] = x_ref[...] + bo_ref[...] + op

    @pl.when(hh > 0)
    def _():
        x2_ref[...] += op

    @pl.when(hh == H - 1)
    def _():
        t = _ln_rows(x2_ref[...], g2_ref[...], b2_ref[...])
        tb_ref[...] = t.astype(jnp.bfloat16)
        lgt_ref[...] = lax.dot_general(wrt_ref[...], t, (((1,), (1,)), ((), ())),
                                       preferred_element_type=jnp.float32)


def _attn_call(x, q, k, v, Wo, bo, ln2_g, ln2_b, WrT, interpret=False):
    S = x.shape[0]
    full = lambda shp: pl.BlockSpec(shp, lambda s, h: (0,) * len(shp))
    return pl.pallas_call(
        _attn_body,
        grid=(S // SB, H),
        in_specs=[
            pl.BlockSpec((SB, D), lambda s, h: (s, 0)),
            pl.BlockSpec((1, SB, DH), lambda s, h: (h, s, 0)),
            pl.BlockSpec((1, S, DH), lambda s, h: (h, 0, 0)),
            pl.BlockSpec((1, S, DH), lambda s, h: (h, 0, 0)),
            pl.BlockSpec((DH, D), lambda s, h: (h, 0)),
            full((1, D)), full((1, D)), full((1, D)),
            full((E, D)),
        ],
        out_specs=[
            pl.BlockSpec((SB, D), lambda s, h: (s, 0)),
            pl.BlockSpec((SB, D), lambda s, h: (s, 0)),
            pl.BlockSpec((E, SB), lambda s, h: (0, s)),
        ],
        out_shape=[
            jax.ShapeDtypeStruct((S, D), jnp.float32),
            jax.ShapeDtypeStruct((S, D), jnp.bfloat16),
            jax.ShapeDtypeStruct((E, S), jnp.float32),
        ],
        compiler_params=pltpu.CompilerParams(
            dimension_semantics=("parallel", "arbitrary")),
        interpret=interpret,
    )(x, q, k, v, Wo, bo, ln2_g, ln2_b, WrT)


def _moe_body(be_ref, st_ref, sg_ref, t_ref, w1_ref, b1_ref, w2_ref, b2_ref,
              y_ref, x_scr, acc_scr):
    S = t_ref.shape[0]
    f = pl.program_id(1)

    @pl.when(f == 0)
    def _():
        st = st_ref[...].astype(jnp.float32)
        iot = lax.broadcasted_iota(jnp.int32, (BT, S), 1).astype(jnp.float32)
        oh = jnp.where(st == iot,
                       jnp.float32(1), jnp.float32(0)).astype(jnp.bfloat16)
        x_scr[...] = jnp.dot(oh, t_ref[...],
                             preferred_element_type=jnp.float32).astype(jnp.bfloat16)
        acc_scr[...] = jnp.broadcast_to(b2_ref[0], (BT, D))

    h1 = jnp.dot(x_scr[...], w1_ref[0].astype(jnp.bfloat16),
                 preferred_element_type=jnp.float32) + b1_ref[0]
    h1 = 0.5 * h1 * (1.0 + lax.erf(h1 * (2 ** -0.5)))
    acc_scr[...] += jnp.dot(h1.astype(jnp.bfloat16), w2_ref[0].astype(jnp.bfloat16),
                            preferred_element_type=jnp.float32)

    @pl.when(f == NF - 1)
    def _():
        y_ref[...] = acc_scr[...] * sg_ref[...]


def _moe_call(blk_e, st, sg, tbf, W1, b1, W2, b2, P, interpret=False):
    S = tbf.shape[0]
    NB = P // BT
    grid_spec = pltpu.PrefetchScalarGridSpec(
        num_scalar_prefetch=1,
        grid=(NB, NF),
        in_specs=[
            pl.BlockSpec((BT, 1), lambda b, f, be: (b, 0)),
            pl.BlockSpec((BT, 1), lambda b, f, be: (b, 0)),
            pl.BlockSpec((S, D), lambda b, f, be: (0, 0)),
            pl.BlockSpec((1, D, FB), lambda b, f, be: (be[b], 0, f)),
            pl.BlockSpec((1, 1, FB), lambda b, f, be: (be[b], 0, f)),
            pl.BlockSpec((1, FB, D), lambda b, f, be: (be[b], f, 0)),
            pl.BlockSpec((1, 1, D), lambda b, f, be: (be[b], 0, 0)),
        ],
        out_specs=pl.BlockSpec((BT, D), lambda b, f, be: (b, 0)),
        scratch_shapes=[
            pltpu.VMEM((BT, D), jnp.bfloat16),
            pltpu.VMEM((BT, D), jnp.float32),
        ],
    )
    return pl.pallas_call(
        _moe_body,
        grid_spec=grid_spec,
        out_shape=jax.ShapeDtypeStruct((P, D), jnp.float32),
        compiler_params=pltpu.CompilerParams(
            dimension_semantics=("arbitrary", "arbitrary")),
        interpret=interpret,
    )(blk_e, st, sg, tbf, W1, b1, W2, b2)


def _route_jnp(lgT, N, P):
    logits = lgT.T
    probs = jax.nn.softmax(logits, axis=-1)
    i1 = jnp.argmax(probs, axis=-1)
    p1 = jnp.max(probs, axis=-1)
    masked = jnp.where(jax.nn.one_hot(i1, E, dtype=bool), -jnp.inf, probs)
    i2 = jnp.argmax(masked, axis=-1)
    p2 = jnp.max(masked, axis=-1)
    g1 = p1 / (p1 + p2)
    g2 = p2 / (p1 + p2)
    eall = jnp.concatenate([i1, i2])
    gall = jnp.concatenate([g1, g2])
    tall = jnp.concatenate([jnp.arange(N), jnp.arange(N)])
    oh = jax.nn.one_hot(eall, E, dtype=jnp.int32)
    counts = jnp.sum(oh, axis=0)
    pc = ((counts + BT - 1) // BT) * BT
    pad_end = jnp.cumsum(pc)
    pad_off = pad_end - pc
    rank = jnp.cumsum(oh, axis=0) - oh
    rank = jnp.take_along_axis(rank, eall[:, None], axis=1)[:, 0]
    pos = pad_off[eall] + rank
    slot_token = jnp.zeros((P,), jnp.int32).at[pos].set(tall)
    slot_gate = jnp.zeros((P,), jnp.float32).at[pos].set(gall)
    NBb = P // BT
    bstart = jnp.arange(NBb) * BT
    blk_e = jnp.sum((bstart[:, None] >= pad_end[None, :]).astype(jnp.int32), axis=1)
    blk_e = jnp.minimum(blk_e, E - 1)
    frac = counts.astype(jnp.float32) / (N * K)
    pmean = jnp.mean(probs, axis=0)
    lb = jnp.float32(0.01) * E * jnp.sum(frac * pmean)
    return slot_token, slot_gate, blk_e, pos[:N], pos[N:], lb


def kernel(x, ln1_g, ln1_b, ln2_g, ln2_b, Wq, bq, Wk, bk, Wv, bv, Wo, bo,
           Wr, W1, b1, W2, b2, interpret=False):
    B, S, _ = x.shape
    N = B * S
    P = K * N + E * BT
    x2d = x.reshape(N, D)
    r1 = lambda a: a.reshape(1, D)
    q, k, v = _qkv_call(x2d, r1(ln1_g), r1(ln1_b), Wq, r1(bq), Wk, r1(bk),
                        Wv, r1(bv), interpret=interpret)
    x2, tbf, lgT = _attn_call(x2d, q, k, v, Wo, r1(bo), r1(ln2_g), r1(ln2_b),
                              Wr.T, interpret=interpret)
    st, sg, blk_e, pos0, pos1, lb = _route_jnp(lgT, N, P)
    y = _moe_call(blk_e, st.reshape(P, 1), sg.reshape(P, 1), tbf,
                  W1, b1.reshape(E, 1, F), W2, b2.reshape(E, 1, D), P,
                  interpret=interpret)
    moe = y[pos0] + y[pos1]
    out = (x2 + moe).reshape(B, S, D)
    return out, lb
